# Initial kernel scaffold; baseline (speedup 1.0000x reference)
import jax
import jax.numpy as jnp
from jax.experimental import pallas as pl


def kernel(feat, edge_index, e_feat, W_fc, edge_emb, W_fc_e, attn_l, attn_r, attn_e, W_res):
    raise NotImplementedError("write your pallas kernel here")


# speedup vs baseline: 19.7618x; 19.7618x over previous
import functools

import jax
import jax.numpy as jnp
from jax import lax
from jax.experimental import pallas as pl
from jax.experimental.pallas import tpu as pltpu
from jax.experimental.pallas import tpu_sc as plsc

N = 10000
E = 320000
IN_FEATS = 128
OUT = 32
H = 8
EF = 16
NEG_SLOPE = 0.2

EPQ = E // 4
CHUNK = 640
NCH = EPQ // CHUNK
NB = N // 16


def _tc_body(feat_b, wfc, wres, al_m, ar_m, emb, wfce, ae_m,
             fs_o, el_o, er_o, res_o, ee_o):
    fs = jnp.dot(feat_b[:], wfc[:], preferred_element_type=jnp.float32)
    fs_o[:] = fs
    el_o[:] = jnp.dot(fs, al_m[:], preferred_element_type=jnp.float32)
    er_o[:] = jnp.dot(fs, ar_m[:], preferred_element_type=jnp.float32)
    res_o[:] = jnp.dot(feat_b[:], wres[:], preferred_element_type=jnp.float32)

    @pl.when(pl.program_id(0) == 0)
    def _():
        ef = jnp.dot(emb[:], wfce[:], preferred_element_type=jnp.float32)
        ee_o[:] = jnp.dot(ef, ae_m[:], preferred_element_type=jnp.float32)


def _sc_body(src1, dst1, et1, elT, erT, eeT, featT, resT,
             aT, rstT, den_hbm,
             el_v, er_v, ee_v, den_v, tmp_v,
             src_v, dst_v, et_v, s_v, a_v, dstix_v, rows_v,
             rst_sh, sem):
    c = lax.axis_index("c")
    s = lax.axis_index("s")
    hl = s // 4
    q = s % 4
    h = c * 4 + hl
    ebase = q * EPQ

    def al8(x):
        return pl.multiple_of(x, 8)

    pltpu.sync_copy(elT.at[pl.ds(al8(h * N), N)], el_v)
    pltpu.sync_copy(erT.at[pl.ds(al8(h * N), N)], er_v)
    pltpu.sync_copy(eeT.at[pl.ds(al8(h * 16), 16)], ee_v)

    zeros16 = jnp.zeros((16,), jnp.float32)

    def _zero(i, carry):
        den_v[pl.ds(i * 16, 16)] = zeros16
        return carry

    lax.fori_loop(0, NB, _zero, 0)

    def _pass1(cidx, carry):
        eoff = ebase + cidx * CHUNK
        pltpu.sync_copy(src1.at[pl.ds(al8(eoff), CHUNK)], src_v)
        pltpu.sync_copy(dst1.at[pl.ds(al8(eoff), CHUNK)], dst_v)
        pltpu.sync_copy(et1.at[pl.ds(al8(eoff), CHUNK)], et_v)

        def _grp(g, carry2):
            si = src_v[pl.ds(g * 16, 16)]
            di = dst_v[pl.ds(g * 16, 16)]
            ti = et_v[pl.ds(g * 16, 16)]
            e = (plsc.load_gather(el_v, [si])
                 + plsc.load_gather(er_v, [di])
                 + plsc.load_gather(ee_v, [ti]))
            e = jnp.where(e > 0, e, NEG_SLOPE * e)
            sv = jnp.exp(e)
            s_v[pl.ds(g * 16, 16)] = sv
            plsc.addupdate_scatter(den_v, [di], sv)
            return carry2

        lax.fori_loop(0, CHUNK // 16, _grp, 0)
        pltpu.sync_copy(s_v, aT.at[pl.ds(al8(h * E + eoff), CHUNK)])
        return carry

    lax.fori_loop(0, NCH, _pass1, 0)

    pltpu.sync_copy(den_v, den_hbm.at[c * 16 + s])

    @pl.when(q == 0)
    def _():
        pltpu.sync_copy(resT.at[h], rst_sh.at[hl])

    plsc.subcore_barrier()

    for qq in (1, 2, 3):
        other = c * 16 + hl * 4 + ((q + qq) % 4)

        def _piece(b, carry):
            pltpu.sync_copy(den_hbm.at[other, pl.ds(b * 2000, 2000)], tmp_v)

            def _acc(i, carry2):
                o = b * 2000 + i * 16
                den_v[pl.ds(o, 16)] = (den_v[pl.ds(o, 16)]
                                       + tmp_v[pl.ds(i * 16, 16)])
                return carry2

            return lax.fori_loop(0, 125, _acc, carry)

        lax.fori_loop(0, 5, _piece, 0)

    def _pass2(cidx, carry):
        eoff = ebase + cidx * CHUNK
        pltpu.sync_copy(src1.at[pl.ds(al8(eoff), CHUNK)], src_v)
        pltpu.sync_copy(aT.at[pl.ds(al8(h * E + eoff), CHUNK)], s_v)

        def _sub(j, carry2):
            pltpu.sync_copy(dst1.at[pl.ds(al8(eoff + j * 128), 128)], dstix_v)

            def _grp(g, carry3):
                di = dstix_v[pl.ds(g * 16, 16)]
                dd = plsc.load_gather(den_v, [di])
                k = j * 128 + g * 16
                av = s_v[pl.ds(k, 16)] / dd
                a_v[pl.ds(k, 16)] = av
                return carry3

            lax.fori_loop(0, 8, _grp, 0)

            pltpu.async_copy(featT.at[h].at[src_v.at[pl.ds(j * 128, 128)]],
                             rows_v, sem).wait()

            def _scale(i, carry3):
                ai = plsc.load_gather(
                    a_v, [jnp.full((16,), j * 128 + i, jnp.int32)])
                rows_v[i, pl.ds(0, 16)] = rows_v[i, pl.ds(0, 16)] * ai
                rows_v[i, pl.ds(16, 16)] = rows_v[i, pl.ds(16, 16)] * ai
                return carry3

            lax.fori_loop(0, 128, _scale, 0)
            pltpu.sync_copy(rows_v, rst_sh.at[hl].at[dstix_v], add=True)
            return carry2

        lax.fori_loop(0, CHUNK // 128, _sub, 0)
        pltpu.sync_copy(a_v, aT.at[pl.ds(al8(h * E + eoff), CHUNK)])
        return carry

    lax.fori_loop(0, NCH, _pass2, 0)

    plsc.subcore_barrier()

    @pl.when(q < 2)
    def _():
        off = al8(q * 2504)
        pltpu.sync_copy(rst_sh.at[hl, pl.ds(off, 2504)],
                        rstT.at[h, pl.ds(off, 2504)])

    @pl.when(q >= 2)
    def _():
        off = al8(5008 + (q - 2) * 2496)
        pltpu.sync_copy(rst_sh.at[hl, pl.ds(off, 2496)],
                        rstT.at[h, pl.ds(off, 2496)])


@jax.jit
def kernel(feat, edge_index, e_feat, W_fc, edge_emb, W_fc_e,
           attn_l, attn_r, attn_e, W_res):
    f32 = jnp.float32

    head_of = jnp.arange(H * OUT) // OUT
    sel = (head_of[:, None] == jnp.arange(H)[None, :]).astype(f32)
    al_m = sel * attn_l[0].reshape(H * OUT)[:, None]
    ar_m = sel * attn_r[0].reshape(H * OUT)[:, None]
    head_of_e = jnp.arange(H * EF) // EF
    sel_e = (head_of_e[:, None] == jnp.arange(H)[None, :]).astype(f32)
    ae_m = sel_e * attn_e[0].reshape(H * EF)[:, None]

    bn = 1000
    full = lambda shape: pl.BlockSpec(shape, lambda i: (0,) * len(shape))
    fs, el, er, res, ee = pl.pallas_call(
        _tc_body,
        grid=(N // bn,),
        in_specs=[
            pl.BlockSpec((bn, IN_FEATS), lambda i: (i, 0)),
            full((IN_FEATS, H * OUT)),
            full((IN_FEATS, H * OUT)),
            full((H * OUT, H)),
            full((H * OUT, H)),
            full((H, EF)),
            full((EF, H * EF)),
            full((H * EF, H)),
        ],
        out_specs=[
            pl.BlockSpec((bn, H * OUT), lambda i: (i, 0)),
            pl.BlockSpec((bn, H), lambda i: (i, 0)),
            pl.BlockSpec((bn, H), lambda i: (i, 0)),
            pl.BlockSpec((bn, H * OUT), lambda i: (i, 0)),
            full((H, H)),
        ],
        out_shape=[
            jax.ShapeDtypeStruct((N, H * OUT), f32),
            jax.ShapeDtypeStruct((N, H), f32),
            jax.ShapeDtypeStruct((N, H), f32),
            jax.ShapeDtypeStruct((N, H * OUT), f32),
            jax.ShapeDtypeStruct((H, H), f32),
        ],
    )(feat, W_fc, W_res, al_m, ar_m, edge_emb, W_fc_e, ae_m)

    src1 = edge_index[0]
    dst1 = edge_index[1]
    et1 = e_feat
    elT = el.T.reshape(H * N)
    erT = er.T.reshape(H * N)
    eeT = jnp.pad(ee.T, ((0, 0), (0, 8))).reshape(H * 16)
    featT = fs.reshape(N, H, OUT).transpose(1, 0, 2)
    resT = res.reshape(N, H, OUT).transpose(1, 0, 2)

    mesh = plsc.VectorSubcoreMesh(core_axis_name="c", subcore_axis_name="s",
                                  num_cores=2, num_subcores=16)
    aT_rst = pl.kernel(
        _sc_body,
        out_type=[
            jax.ShapeDtypeStruct((H * E,), f32),
            jax.ShapeDtypeStruct((H, N, OUT), f32),
            jax.ShapeDtypeStruct((32, N), f32),
        ],
        mesh=mesh,
        compiler_params=pltpu.CompilerParams(needs_layout_passes=False,
                                             use_tc_tiling_on_sc=False),
        scratch_types=[
            pltpu.VMEM((N,), f32),
            pltpu.VMEM((N,), f32),
            pltpu.VMEM((16,), f32),
            pltpu.VMEM((N,), f32),
            pltpu.VMEM((2000,), f32),
            pltpu.VMEM((CHUNK,), jnp.int32),
            pltpu.VMEM((CHUNK,), jnp.int32),
            pltpu.VMEM((CHUNK,), jnp.int32),
            pltpu.VMEM((CHUNK,), f32),
            pltpu.VMEM((CHUNK,), f32),
            pltpu.VMEM((128,), jnp.int32),
            pltpu.VMEM((128, OUT), f32),
            pltpu.VMEM_SHARED((4, N, OUT), f32),
            pltpu.SemaphoreType.DMA,
        ],
    )(src1, dst1, et1, elT, erT, eeT, featT, resT)

    aT, rstT = aT_rst[0], aT_rst[1]
    a = aT.reshape(H, E).T
    rst = rstT.transpose(1, 0, 2)
    return rst, a


# speedup vs baseline: 26.6681x; 1.3495x over previous
import functools

import jax
import jax.numpy as jnp
from jax import lax
from jax.experimental import pallas as pl
from jax.experimental.pallas import tpu as pltpu
from jax.experimental.pallas import tpu_sc as plsc

N = 10000
E = 320000
IN_FEATS = 128
OUT = 32
H = 8
EF = 16
NEG_SLOPE = 0.2

EPQ = E // 4
ROWS = E // 128
RPQ = ROWS // 4
STG = 5
NCH = RPQ // STG
NB = N // 16


def _tc_body(feat_b, wfc, wres, al_m, ar_m, emb, wfce, ae_m,
             fs_o, el_o, er_o, res_o, ee_o):
    fs = jnp.dot(feat_b[:], wfc[:], preferred_element_type=jnp.float32)
    fs_o[:] = fs
    el_o[:] = jnp.dot(fs, al_m[:], preferred_element_type=jnp.float32)
    er_o[:] = jnp.dot(fs, ar_m[:], preferred_element_type=jnp.float32)
    res_o[:] = jnp.dot(feat_b[:], wres[:], preferred_element_type=jnp.float32)

    @pl.when(pl.program_id(0) == 0)
    def _():
        ef = jnp.dot(emb[:], wfce[:], preferred_element_type=jnp.float32)
        ee_o[:] = jnp.dot(ef, ae_m[:], preferred_element_type=jnp.float32)


def _sc_body(src2, dst2, et2, elT, erT, eeT, featT, resT,
             aT, rstT, den_hbm,
             el_v, er_v, ee_v, den_v, tmp_v,
             src_st, dst_st, et_st, s_st, a_st, rows3,
             rst_sh, sem_ld0, sem_ld1, sem_sw0, sem_sw1,
             sem_g0, sem_g1, sem_g2, sem_sc0, sem_sc1, sem_sc2):
    sem_ld = (sem_ld0, sem_ld1)
    sem_sw = (sem_sw0, sem_sw1)
    sem_g = (sem_g0, sem_g1, sem_g2)
    sem_sc = (sem_sc0, sem_sc1, sem_sc2)

    def route(sems, idx, fn):
        if isinstance(idx, int):
            fn(sems[idx], idx)
            return
        for k in range(len(sems)):
            @pl.when(idx == k)
            def _(k=k):
                fn(sems[k], k)
    c = lax.axis_index("c")
    s = lax.axis_index("s")
    hl = s // 4
    q = s % 4
    h = c * 4 + hl
    rbase = q * RPQ

    def al8(x):
        return pl.multiple_of(x, 8)

    pltpu.sync_copy(elT.at[pl.ds(al8(h * N), N)], el_v)
    pltpu.sync_copy(erT.at[pl.ds(al8(h * N), N)], er_v)
    pltpu.sync_copy(eeT.at[pl.ds(al8(h * 16), 16)], ee_v)

    zeros16 = jnp.zeros((16,), jnp.float32)

    def _zero(i, carry):
        den_v[pl.ds(i * 16, 16)] = zeros16
        return carry

    lax.fori_loop(0, NB, _zero, 0)

    def p1_issue(ci, b):
        roff = rbase + ci * STG

        def go(sem, k):
            dsts = pl.ds(k * STG, STG)
            pltpu.async_copy(src2.at[pl.ds(roff, STG)], src_st.at[dsts], sem)
            pltpu.async_copy(dst2.at[pl.ds(roff, STG)], dst_st.at[dsts], sem)
            pltpu.async_copy(et2.at[pl.ds(roff, STG)], et_st.at[dsts], sem)

        route(sem_ld, b, go)

    def p1_wait(b):
        def go(sem, k):
            dsts = pl.ds(k * STG, STG)
            for hb, vb in ((src2, src_st), (dst2, dst_st), (et2, et_st)):
                pltpu.make_async_copy(hb.at[pl.ds(0, STG)], vb.at[dsts],
                                      sem).wait()

        route(sem_ld, b, go)

    def sw_issue(ci, b):
        arow = h * ROWS + rbase + ci * STG

        def go(sem, k):
            pltpu.async_copy(s_st.at[pl.ds(k * STG, STG)],
                             aT.at[pl.ds(arow, STG)], sem)

        route(sem_sw, b, go)

    def sw_wait(b):
        def go(sem, k):
            pltpu.make_async_copy(s_st.at[pl.ds(k * STG, STG)],
                                  aT.at[pl.ds(0, STG)], sem).wait()

        route(sem_sw, b, go)

    p1_issue(0, 0)

    def _p1(ci, carry):
        b = ci % 2

        @pl.when(ci + 1 < NCH)
        def _():
            p1_issue(ci + 1, 1 - b)

        p1_wait(b)

        @pl.when(ci >= 2)
        def _():
            sw_wait(b)

        def _row(r, carry2):
            row = b * STG + r

            def _grp(g, carry3):
                si = src_st[row, pl.ds(g * 16, 16)]
                di = dst_st[row, pl.ds(g * 16, 16)]
                ti = et_st[row, pl.ds(g * 16, 16)]
                e = (plsc.load_gather(el_v, [si])
                     + plsc.load_gather(er_v, [di])
                     + plsc.load_gather(ee_v, [ti]))
                e = jnp.where(e > 0, e, NEG_SLOPE * e)
                sv = jnp.exp(e)
                s_st[row, pl.ds(g * 16, 16)] = sv
                plsc.addupdate_scatter(den_v, [di], sv)
                return carry3

            return lax.fori_loop(0, 8, _grp, carry2)

        lax.fori_loop(0, STG, _row, 0)
        sw_issue(ci, b)
        return carry

    lax.fori_loop(0, NCH, _p1, 0)
    sw_wait(0)
    sw_wait(1)

    pltpu.sync_copy(den_v, den_hbm.at[c * 16 + s])

    @pl.when(q == 0)
    def _():
        pltpu.sync_copy(resT.at[h], rst_sh.at[hl])

    plsc.subcore_barrier()

    for qq in (1, 2, 3):
        other = c * 16 + hl * 4 + ((q + qq) % 4)

        def _piece(b, carry):
            pltpu.sync_copy(den_hbm.at[other, pl.ds(b * 2000, 2000)], tmp_v)

            def _acc(i, carry2):
                o = b * 2000 + i * 16
                den_v[pl.ds(o, 16)] = (den_v[pl.ds(o, 16)]
                                       + tmp_v[pl.ds(i * 16, 16)])
                return carry2

            return lax.fori_loop(0, 125, _acc, carry)

        lax.fori_loop(0, 5, _piece, 0)

    def p2_issue(ci, b):
        roff = rbase + ci * STG
        arow = h * ROWS + roff

        def go(sem, k):
            dsts = pl.ds(k * STG, STG)
            pltpu.async_copy(src2.at[pl.ds(roff, STG)], src_st.at[dsts], sem)
            pltpu.async_copy(dst2.at[pl.ds(roff, STG)], dst_st.at[dsts], sem)
            pltpu.async_copy(aT.at[pl.ds(arow, STG)], s_st.at[dsts], sem)

        route(sem_ld, b, go)

    def p2_wait(b):
        def go(sem, k):
            dsts = pl.ds(k * STG, STG)
            pltpu.make_async_copy(src2.at[pl.ds(0, STG)], src_st.at[dsts],
                                  sem).wait()
            pltpu.make_async_copy(dst2.at[pl.ds(0, STG)], dst_st.at[dsts],
                                  sem).wait()
            pltpu.make_async_copy(aT.at[pl.ds(0, STG)], s_st.at[dsts],
                                  sem).wait()

        route(sem_ld, b, go)

    def aw_issue(ci, b):
        arow = h * ROWS + rbase + ci * STG

        def go(sem, k):
            pltpu.async_copy(a_st.at[pl.ds(k * STG, STG)],
                             aT.at[pl.ds(arow, STG)], sem)

        route(sem_sw, b, go)

    def aw_wait(b):
        def go(sem, k):
            pltpu.make_async_copy(a_st.at[pl.ds(k * STG, STG)],
                                  aT.at[pl.ds(0, STG)], sem).wait()

        route(sem_sw, b, go)

    def g_issue(row, x):
        def go(sem, k):
            pltpu.async_copy(featT.at[h].at[src_st.at[row]], rows3.at[k],
                             sem)

        route(sem_g, x, go)

    def g_wait(x):
        def go(sem, k):
            pltpu.make_async_copy(featT.at[h].at[src_st.at[0]],
                                  rows3.at[k], sem).wait()

        route(sem_g, x, go)

    def sc_issue(row, x):
        def go(sem, k):
            pltpu.async_copy(rows3.at[k], rst_sh.at[hl].at[dst_st.at[row]],
                             sem, add=True)

        route(sem_sc, x, go)

    def sc_wait(x):
        def go(sem, k):
            pltpu.make_async_copy(rows3.at[k], rst_sh.at[hl].at[dst_st.at[0]],
                                  sem).wait()

        route(sem_sc, x, go)

    p2_issue(0, 0)

    def _p2(ci, carry):
        b = ci % 2
        p2_wait(b)

        @pl.when(ci >= 2)
        def _():
            aw_wait(b)

        def _row(r, carry2):
            row = b * STG + r

            def _grp(g, carry3):
                di = dst_st[row, pl.ds(g * 16, 16)]
                dd = plsc.load_gather(den_v, [di])
                av = s_st[row, pl.ds(g * 16, 16)] / dd
                a_st[row, pl.ds(g * 16, 16)] = av
                return carry3

            return lax.fori_loop(0, 8, _grp, carry2)

        lax.fori_loop(0, STG, _row, 0)
        aw_issue(ci, b)

        t0 = ci * STG

        @pl.when(t0 >= 3)
        def _():
            sc_wait(t0 % 3)

        g_issue(b * STG, t0 % 3)

        def _sub(j, carry2):
            t = t0 + j
            x = t % 3

            @pl.when(j + 1 < STG)
            def _():
                xn = (t + 1) % 3

                @pl.when(t + 1 >= 3)
                def _():
                    sc_wait(xn)

                g_issue(b * STG + j + 1, xn)

            g_wait(x)

            def _scale(i, carry3):
                ai = plsc.load_gather(
                    a_st, [jnp.full((16,), b * STG + j, jnp.int32),
                           jnp.full((16,), i, jnp.int32)])
                rows3[x, i, pl.ds(0, 16)] = rows3[x, i, pl.ds(0, 16)] * ai
                rows3[x, i, pl.ds(16, 16)] = rows3[x, i, pl.ds(16, 16)] * ai
                return carry3

            lax.fori_loop(0, 128, _scale, 0)
            sc_issue(b * STG + j, x)
            return carry2

        lax.fori_loop(0, STG, _sub, 0)

        @pl.when(ci + 1 < NCH)
        def _():
            p2_issue(ci + 1, 1 - b)

        return carry

    lax.fori_loop(0, NCH, _p2, 0)
    aw_wait(0)
    aw_wait(1)
    sc_wait(1)
    sc_wait(2)
    sc_wait(0)

    plsc.subcore_barrier()

    @pl.when(q < 2)
    def _():
        off = al8(q * 2504)
        pltpu.sync_copy(rst_sh.at[hl, pl.ds(off, 2504)],
                        rstT.at[h, pl.ds(off, 2504)])

    @pl.when(q >= 2)
    def _():
        off = al8(5008 + (q - 2) * 2496)
        pltpu.sync_copy(rst_sh.at[hl, pl.ds(off, 2496)],
                        rstT.at[h, pl.ds(off, 2496)])


@jax.jit
def kernel(feat, edge_index, e_feat, W_fc, edge_emb, W_fc_e,
           attn_l, attn_r, attn_e, W_res):
    f32 = jnp.float32

    head_of = jnp.arange(H * OUT) // OUT
    sel = (head_of[:, None] == jnp.arange(H)[None, :]).astype(f32)
    al_m = sel * attn_l[0].reshape(H * OUT)[:, None]
    ar_m = sel * attn_r[0].reshape(H * OUT)[:, None]
    head_of_e = jnp.arange(H * EF) // EF
    sel_e = (head_of_e[:, None] == jnp.arange(H)[None, :]).astype(f32)
    ae_m = sel_e * attn_e[0].reshape(H * EF)[:, None]

    bn = 1000
    full = lambda shape: pl.BlockSpec(shape, lambda i: (0,) * len(shape))
    fs, el, er, res, ee = pl.pallas_call(
        _tc_body,
        grid=(N // bn,),
        in_specs=[
            pl.BlockSpec((bn, IN_FEATS), lambda i: (i, 0)),
            full((IN_FEATS, H * OUT)),
            full((IN_FEATS, H * OUT)),
            full((H * OUT, H)),
            full((H * OUT, H)),
            full((H, EF)),
            full((EF, H * EF)),
            full((H * EF, H)),
        ],
        out_specs=[
            pl.BlockSpec((bn, H * OUT), lambda i: (i, 0)),
            pl.BlockSpec((bn, H), lambda i: (i, 0)),
            pl.BlockSpec((bn, H), lambda i: (i, 0)),
            pl.BlockSpec((bn, H * OUT), lambda i: (i, 0)),
            full((H, H)),
        ],
        out_shape=[
            jax.ShapeDtypeStruct((N, H * OUT), f32),
            jax.ShapeDtypeStruct((N, H), f32),
            jax.ShapeDtypeStruct((N, H), f32),
            jax.ShapeDtypeStruct((N, H * OUT), f32),
            jax.ShapeDtypeStruct((H, H), f32),
        ],
    )(feat, W_fc, W_res, al_m, ar_m, edge_emb, W_fc_e, ae_m)

    src2 = edge_index[0].reshape(ROWS, 128)
    dst2 = edge_index[1].reshape(ROWS, 128)
    et2 = e_feat.reshape(ROWS, 128)
    elT = el.T.reshape(H * N)
    erT = er.T.reshape(H * N)
    eeT = jnp.pad(ee.T, ((0, 0), (0, 8))).reshape(H * 16)
    featT = fs.reshape(N, H, OUT).transpose(1, 0, 2)
    resT = res.reshape(N, H, OUT).transpose(1, 0, 2)

    mesh = plsc.VectorSubcoreMesh(core_axis_name="c", subcore_axis_name="s",
                                  num_cores=2, num_subcores=16)
    aT_rst = pl.kernel(
        _sc_body,
        out_type=[
            jax.ShapeDtypeStruct((H * ROWS, 128), f32),
            jax.ShapeDtypeStruct((H, N, OUT), f32),
            jax.ShapeDtypeStruct((32, N), f32),
        ],
        mesh=mesh,
        compiler_params=pltpu.CompilerParams(needs_layout_passes=False,
                                             use_tc_tiling_on_sc=False),
        scratch_types=[
            pltpu.VMEM((N,), f32),
            pltpu.VMEM((N,), f32),
            pltpu.VMEM((16,), f32),
            pltpu.VMEM((N,), f32),
            pltpu.VMEM((2000,), f32),
            pltpu.VMEM((2 * STG, 128), jnp.int32),
            pltpu.VMEM((2 * STG, 128), jnp.int32),
            pltpu.VMEM((2 * STG, 128), jnp.int32),
            pltpu.VMEM((2 * STG, 128), f32),
            pltpu.VMEM((2 * STG, 128), f32),
            pltpu.VMEM((3, 128, OUT), f32),
            pltpu.VMEM_SHARED((4, N, OUT), f32),
        ] + [pltpu.SemaphoreType.DMA] * 10,
    )(src2, dst2, et2, elT, erT, eeT, featT, resT)

    aT, rstT = aT_rst[0], aT_rst[1]
    a = aT.reshape(H, E).T
    rst = rstT.transpose(1, 0, 2)
    return rst, a


# speedup vs baseline: 38.6718x; 1.4501x over previous
import functools

import jax
import jax.numpy as jnp
from jax import lax
from jax.experimental import pallas as pl
from jax.experimental.pallas import tpu as pltpu
from jax.experimental.pallas import tpu_sc as plsc

N = 10000
E = 320000
IN_FEATS = 128
OUT = 32
H = 8
EF = 16
NEG_SLOPE = 0.2

EPQ = E // 4
ROWS = E // 128
RPQ = ROWS // 4
STG = 5
NCH = RPQ // STG
NB = N // 16


def _tc_body(feat_b, wfc, wres, al_m, ar_m, emb, wfce, ae_m,
             fs_o, el_o, er_o, res_o, ee_o):
    fs = jnp.dot(feat_b[:], wfc[:], preferred_element_type=jnp.float32)
    fs_o[:] = fs
    el_o[:] = jnp.dot(fs, al_m[:], preferred_element_type=jnp.float32)
    er_o[:] = jnp.dot(fs, ar_m[:], preferred_element_type=jnp.float32)
    res_o[:] = jnp.dot(feat_b[:], wres[:], preferred_element_type=jnp.float32)

    @pl.when(pl.program_id(0) == 0)
    def _():
        ef = jnp.dot(emb[:], wfce[:], preferred_element_type=jnp.float32)
        ee_o[:] = jnp.dot(ef, ae_m[:], preferred_element_type=jnp.float32)


def _sc_body(src2, dst2, et2, elT, erT, eeT, featT, resT,
             aT, rstT, den_hbm,
             el_v, er_v, ee_v, den_v, tmp_v,
             src_st, dst_st, et_st, s_st, a_st, rows3,
             rst_sh, sem_ld0, sem_ld1, sem_sw0, sem_sw1,
             sem_g0, sem_g1, sem_g2, sem_sc0, sem_sc1, sem_sc2):
    sem_ld = (sem_ld0, sem_ld1)
    sem_sw = (sem_sw0, sem_sw1)
    sem_g = (sem_g0, sem_g1, sem_g2)
    sem_sc = (sem_sc0, sem_sc1, sem_sc2)

    def route(sems, idx, fn):
        if isinstance(idx, int):
            fn(sems[idx], idx)
            return
        for k in range(len(sems)):
            @pl.when(idx == k)
            def _(k=k):
                fn(sems[k], k)
    c = lax.axis_index("c")
    s = lax.axis_index("s")
    hl = s // 4
    q = s % 4
    h = c * 4 + hl
    rbase = q * RPQ

    def al8(x):
        return pl.multiple_of(x, 8)

    pltpu.sync_copy(elT.at[pl.ds(al8(h * N), N)], el_v)
    pltpu.sync_copy(erT.at[pl.ds(al8(h * N), N)], er_v)
    pltpu.sync_copy(eeT.at[pl.ds(al8(h * 16), 16)], ee_v)

    zeros16 = jnp.zeros((16,), jnp.float32)

    def _zero(i, carry):
        den_v[pl.ds(i * 16, 16)] = zeros16
        return carry

    lax.fori_loop(0, NB, _zero, 0)

    def p1_issue(ci, b):
        roff = rbase + ci * STG

        def go(sem, k):
            dsts = pl.ds(k * STG, STG)
            pltpu.async_copy(src2.at[pl.ds(roff, STG)], src_st.at[dsts], sem)
            pltpu.async_copy(dst2.at[pl.ds(roff, STG)], dst_st.at[dsts], sem)
            pltpu.async_copy(et2.at[pl.ds(roff, STG)], et_st.at[dsts], sem)

        route(sem_ld, b, go)

    def p1_wait(b):
        def go(sem, k):
            dsts = pl.ds(k * STG, STG)
            for hb, vb in ((src2, src_st), (dst2, dst_st), (et2, et_st)):
                pltpu.make_async_copy(hb.at[pl.ds(0, STG)], vb.at[dsts],
                                      sem).wait()

        route(sem_ld, b, go)

    def sw_issue(ci, b):
        arow = h * ROWS + rbase + ci * STG

        def go(sem, k):
            pltpu.async_copy(s_st.at[pl.ds(k * STG, STG)],
                             aT.at[pl.ds(arow, STG)], sem)

        route(sem_sw, b, go)

    def sw_wait(b):
        def go(sem, k):
            pltpu.make_async_copy(s_st.at[pl.ds(k * STG, STG)],
                                  aT.at[pl.ds(0, STG)], sem).wait()

        route(sem_sw, b, go)

    p1_issue(0, 0)

    def _p1(ci, carry):
        b = ci % 2

        @pl.when(ci + 1 < NCH)
        def _():
            p1_issue(ci + 1, 1 - b)

        p1_wait(b)

        @pl.when(ci >= 2)
        def _():
            sw_wait(b)

        def _row(r, carry2):
            row = b * STG + r

            def _grp(g, carry3):
                si = src_st[row, pl.ds(g * 16, 16)]
                di = dst_st[row, pl.ds(g * 16, 16)]
                ti = et_st[row, pl.ds(g * 16, 16)]
                e = (plsc.load_gather(el_v, [si])
                     + plsc.load_gather(er_v, [di])
                     + plsc.load_gather(ee_v, [ti]))
                e = jnp.where(e > 0, e, NEG_SLOPE * e)
                sv = jnp.exp(e)
                s_st[row, pl.ds(g * 16, 16)] = sv
                plsc.addupdate_scatter(den_v, [di], sv)
                return carry3

            return lax.fori_loop(0, 8, _grp, carry2, unroll=8)

        lax.fori_loop(0, STG, _row, 0)
        sw_issue(ci, b)
        return carry

    lax.fori_loop(0, NCH, _p1, 0)
    sw_wait(0)
    sw_wait(1)

    pltpu.sync_copy(den_v, den_hbm.at[c * 16 + s])

    @pl.when(q == 0)
    def _():
        pltpu.sync_copy(resT.at[h], rst_sh.at[hl])

    plsc.subcore_barrier()

    for qq in (1, 2, 3):
        other = c * 16 + hl * 4 + ((q + qq) % 4)

        def _piece(b, carry):
            pltpu.sync_copy(den_hbm.at[other, pl.ds(b * 2000, 2000)], tmp_v)

            def _acc(i, carry2):
                o = b * 2000 + i * 16
                den_v[pl.ds(o, 16)] = (den_v[pl.ds(o, 16)]
                                       + tmp_v[pl.ds(i * 16, 16)])
                return carry2

            return lax.fori_loop(0, 125, _acc, carry)

        lax.fori_loop(0, 5, _piece, 0)

    def p2_issue(ci, b):
        roff = rbase + ci * STG
        arow = h * ROWS + roff

        def go(sem, k):
            dsts = pl.ds(k * STG, STG)
            pltpu.async_copy(src2.at[pl.ds(roff, STG)], src_st.at[dsts], sem)
            pltpu.async_copy(dst2.at[pl.ds(roff, STG)], dst_st.at[dsts], sem)
            pltpu.async_copy(aT.at[pl.ds(arow, STG)], s_st.at[dsts], sem)

        route(sem_ld, b, go)

    def p2_wait(b):
        def go(sem, k):
            dsts = pl.ds(k * STG, STG)
            pltpu.make_async_copy(src2.at[pl.ds(0, STG)], src_st.at[dsts],
                                  sem).wait()
            pltpu.make_async_copy(dst2.at[pl.ds(0, STG)], dst_st.at[dsts],
                                  sem).wait()
            pltpu.make_async_copy(aT.at[pl.ds(0, STG)], s_st.at[dsts],
                                  sem).wait()

        route(sem_ld, b, go)

    def aw_issue(ci, b):
        arow = h * ROWS + rbase + ci * STG

        def go(sem, k):
            pltpu.async_copy(a_st.at[pl.ds(k * STG, STG)],
                             aT.at[pl.ds(arow, STG)], sem)

        route(sem_sw, b, go)

    def aw_wait(b):
        def go(sem, k):
            pltpu.make_async_copy(a_st.at[pl.ds(k * STG, STG)],
                                  aT.at[pl.ds(0, STG)], sem).wait()

        route(sem_sw, b, go)

    def g_issue(row, x):
        def go(sem, k):
            pltpu.async_copy(featT.at[h].at[src_st.at[row]], rows3.at[k],
                             sem)

        route(sem_g, x, go)

    def g_wait(x):
        def go(sem, k):
            pltpu.make_async_copy(featT.at[h].at[src_st.at[0]],
                                  rows3.at[k], sem).wait()

        route(sem_g, x, go)

    def sc_issue(row, x):
        def go(sem, k):
            pltpu.async_copy(rows3.at[k], rst_sh.at[hl].at[dst_st.at[row]],
                             sem, add=True)

        route(sem_sc, x, go)

    def sc_wait(x):
        def go(sem, k):
            pltpu.make_async_copy(rows3.at[k], rst_sh.at[hl].at[dst_st.at[0]],
                                  sem).wait()

        route(sem_sc, x, go)

    p2_issue(0, 0)

    def _p2(ci, carry):
        b = ci % 2
        p2_wait(b)

        @pl.when(ci >= 2)
        def _():
            aw_wait(b)

        def _row(r, carry2):
            row = b * STG + r

            def _grp(g, carry3):
                di = dst_st[row, pl.ds(g * 16, 16)]
                dd = plsc.load_gather(den_v, [di])
                av = s_st[row, pl.ds(g * 16, 16)] / dd
                a_st[row, pl.ds(g * 16, 16)] = av
                return carry3

            return lax.fori_loop(0, 8, _grp, carry2, unroll=8)

        lax.fori_loop(0, STG, _row, 0)
        aw_issue(ci, b)

        t0 = ci * STG

        @pl.when(t0 >= 3)
        def _():
            sc_wait(t0 % 3)

        g_issue(b * STG, t0 % 3)

        def _sub(j, carry2):
            t = t0 + j
            x = t % 3

            @pl.when(j + 1 < STG)
            def _():
                xn = (t + 1) % 3

                @pl.when(t + 1 >= 3)
                def _():
                    sc_wait(xn)

                g_issue(b * STG + j + 1, xn)

            g_wait(x)

            def _scale(i, carry3):
                ai = plsc.load_gather(
                    a_st, [jnp.full((16,), b * STG + j, jnp.int32),
                           jnp.full((16,), i, jnp.int32)])
                rows3[x, i, pl.ds(0, 16)] = rows3[x, i, pl.ds(0, 16)] * ai
                rows3[x, i, pl.ds(16, 16)] = rows3[x, i, pl.ds(16, 16)] * ai
                return carry3

            lax.fori_loop(0, 128, _scale, 0, unroll=8)
            sc_issue(b * STG + j, x)
            return carry2

        lax.fori_loop(0, STG, _sub, 0)

        @pl.when(ci + 1 < NCH)
        def _():
            p2_issue(ci + 1, 1 - b)

        return carry

    lax.fori_loop(0, NCH, _p2, 0)
    aw_wait(0)
    aw_wait(1)
    sc_wait(1)
    sc_wait(2)
    sc_wait(0)

    plsc.subcore_barrier()

    @pl.when(q < 2)
    def _():
        off = al8(q * 2504)
        pltpu.sync_copy(rst_sh.at[hl, pl.ds(off, 2504)],
                        rstT.at[h, pl.ds(off, 2504)])

    @pl.when(q >= 2)
    def _():
        off = al8(5008 + (q - 2) * 2496)
        pltpu.sync_copy(rst_sh.at[hl, pl.ds(off, 2496)],
                        rstT.at[h, pl.ds(off, 2496)])


@jax.jit
def kernel(feat, edge_index, e_feat, W_fc, edge_emb, W_fc_e,
           attn_l, attn_r, attn_e, W_res):
    f32 = jnp.float32

    head_of = jnp.arange(H * OUT) // OUT
    sel = (head_of[:, None] == jnp.arange(H)[None, :]).astype(f32)
    al_m = sel * attn_l[0].reshape(H * OUT)[:, None]
    ar_m = sel * attn_r[0].reshape(H * OUT)[:, None]
    head_of_e = jnp.arange(H * EF) // EF
    sel_e = (head_of_e[:, None] == jnp.arange(H)[None, :]).astype(f32)
    ae_m = sel_e * attn_e[0].reshape(H * EF)[:, None]

    bn = 1000
    full = lambda shape: pl.BlockSpec(shape, lambda i: (0,) * len(shape))
    fs, el, er, res, ee = pl.pallas_call(
        _tc_body,
        grid=(N // bn,),
        in_specs=[
            pl.BlockSpec((bn, IN_FEATS), lambda i: (i, 0)),
            full((IN_FEATS, H * OUT)),
            full((IN_FEATS, H * OUT)),
            full((H * OUT, H)),
            full((H * OUT, H)),
            full((H, EF)),
            full((EF, H * EF)),
            full((H * EF, H)),
        ],
        out_specs=[
            pl.BlockSpec((bn, H * OUT), lambda i: (i, 0)),
            pl.BlockSpec((bn, H), lambda i: (i, 0)),
            pl.BlockSpec((bn, H), lambda i: (i, 0)),
            pl.BlockSpec((bn, H * OUT), lambda i: (i, 0)),
            full((H, H)),
        ],
        out_shape=[
            jax.ShapeDtypeStruct((N, H * OUT), f32),
            jax.ShapeDtypeStruct((N, H), f32),
            jax.ShapeDtypeStruct((N, H), f32),
            jax.ShapeDtypeStruct((N, H * OUT), f32),
            jax.ShapeDtypeStruct((H, H), f32),
        ],
    )(feat, W_fc, W_res, al_m, ar_m, edge_emb, W_fc_e, ae_m)

    src2 = edge_index[0].reshape(ROWS, 128)
    dst2 = edge_index[1].reshape(ROWS, 128)
    et2 = e_feat.reshape(ROWS, 128)
    elT = el.T.reshape(H * N)
    erT = er.T.reshape(H * N)
    eeT = jnp.pad(ee.T, ((0, 0), (0, 8))).reshape(H * 16)
    featT = fs.reshape(N, H, OUT).transpose(1, 0, 2)
    resT = res.reshape(N, H, OUT).transpose(1, 0, 2)

    mesh = plsc.VectorSubcoreMesh(core_axis_name="c", subcore_axis_name="s",
                                  num_cores=2, num_subcores=16)
    aT_rst = pl.kernel(
        _sc_body,
        out_type=[
            jax.ShapeDtypeStruct((H * ROWS, 128), f32),
            jax.ShapeDtypeStruct((H, N, OUT), f32),
            jax.ShapeDtypeStruct((32, N), f32),
        ],
        mesh=mesh,
        compiler_params=pltpu.CompilerParams(needs_layout_passes=False,
                                             use_tc_tiling_on_sc=False),
        scratch_types=[
            pltpu.VMEM((N,), f32),
            pltpu.VMEM((N,), f32),
            pltpu.VMEM((16,), f32),
            pltpu.VMEM((N,), f32),
            pltpu.VMEM((2000,), f32),
            pltpu.VMEM((2 * STG, 128), jnp.int32),
            pltpu.VMEM((2 * STG, 128), jnp.int32),
            pltpu.VMEM((2 * STG, 128), jnp.int32),
            pltpu.VMEM((2 * STG, 128), f32),
            pltpu.VMEM((2 * STG, 128), f32),
            pltpu.VMEM((3, 128, OUT), f32),
            pltpu.VMEM_SHARED((4, N, OUT), f32),
        ] + [pltpu.SemaphoreType.DMA] * 10,
    )(src2, dst2, et2, elT, erT, eeT, featT, resT)

    aT, rstT = aT_rst[0], aT_rst[1]
    a = aT.reshape(H, E).T
    rst = rstT.transpose(1, 0, 2)
    return rst, a


# speedup vs baseline: 40.2947x; 1.0420x over previous
import jax
import jax.numpy as jnp
from jax import lax
from jax.experimental import pallas as pl
from jax.experimental.pallas import tpu as pltpu
from jax.experimental.pallas import tpu_sc as plsc

N = 10000
E = 320000
IN_FEATS = 128
OUT = 32
H = 8
EF = 16
NEG_SLOPE = 0.2

ROWS = E // 128
RPQ = ROWS // 4
STG = 5
NCH = RPQ // STG
NB = N // 16


def _tc_body(feat_b, wfc, wres, al_m, ar_m, emb, wfce, ae_m,
             fs_o, el_o, er_o, res_o, ee_o):
    fs = jnp.dot(feat_b[:], wfc[:], preferred_element_type=jnp.float32)
    fs_o[:] = fs
    el_o[:] = jnp.dot(fs, al_m[:], preferred_element_type=jnp.float32)
    er_o[:] = jnp.dot(fs, ar_m[:], preferred_element_type=jnp.float32)
    res_o[:] = jnp.dot(feat_b[:], wres[:], preferred_element_type=jnp.float32)

    @pl.when(pl.program_id(0) == 0)
    def _():
        ef = jnp.dot(emb[:], wfce[:], preferred_element_type=jnp.float32)
        ee_o[:] = jnp.dot(ef, ae_m[:], preferred_element_type=jnp.float32)


def _sc_body(src2, dst2, et2, elT, erT, eeT, featT, resT,
             aT, rstT, den_hbm,
             el_v, er_v, ee_v, den_v, tmp_v,
             src_st, dst_st, et_st, s_st, a_st, rows4,
             rst_sh,
             sem_ld0, sem_ld1, sem_ld2, sem_sw0, sem_sw1, sem_sw2,
             sem_g0, sem_g1, sem_g2, sem_g3,
             sem_sc0, sem_sc1, sem_sc2, sem_sc3):
    sem_ld = (sem_ld0, sem_ld1, sem_ld2)
    sem_sw = (sem_sw0, sem_sw1, sem_sw2)
    sem_g = (sem_g0, sem_g1, sem_g2, sem_g3)
    sem_sc = (sem_sc0, sem_sc1, sem_sc2, sem_sc3)

    def route(sems, idx, fn):
        if isinstance(idx, int):
            fn(sems[idx], idx)
            return
        for k in range(len(sems)):
            @pl.when(idx == k)
            def _(k=k):
                fn(sems[k], k)

    c = lax.axis_index("c")
    s = lax.axis_index("s")
    hl = s // 4
    q = s % 4
    h = c * 4 + hl
    rbase = q * RPQ

    def al8(x):
        return pl.multiple_of(x, 8)

    pltpu.sync_copy(elT.at[pl.ds(al8(h * N), N)], el_v)
    pltpu.sync_copy(erT.at[pl.ds(al8(h * N), N)], er_v)
    pltpu.sync_copy(eeT.at[pl.ds(al8(h * 16), 16)], ee_v)

    zeros16 = jnp.zeros((16,), jnp.float32)

    def _zero(i, carry):
        den_v[pl.ds(i * 16, 16)] = zeros16
        return carry

    lax.fori_loop(0, NB, _zero, 0)

    def p1_issue(ci):
        roff = rbase + ci * STG

        def go(sem, k):
            dsts = pl.ds(k * STG, STG)
            pltpu.async_copy(src2.at[pl.ds(roff, STG)], src_st.at[dsts], sem)
            pltpu.async_copy(dst2.at[pl.ds(roff, STG)], dst_st.at[dsts], sem)
            pltpu.async_copy(et2.at[pl.ds(roff, STG)], et_st.at[dsts], sem)

        route(sem_ld, ci % 3, go)

    def p1_wait(b):
        def go(sem, k):
            dsts = pl.ds(k * STG, STG)
            for hb, vb in ((src2, src_st), (dst2, dst_st), (et2, et_st)):
                pltpu.make_async_copy(hb.at[pl.ds(0, STG)], vb.at[dsts],
                                      sem).wait()

        route(sem_ld, b, go)

    def sw_issue(ci, b):
        arow = h * ROWS + rbase + ci * STG

        def go(sem, k):
            pltpu.async_copy(s_st.at[pl.ds(k * STG, STG)],
                             aT.at[pl.ds(arow, STG)], sem)

        route(sem_sw, b, go)

    def sw_wait(b):
        def go(sem, k):
            pltpu.make_async_copy(s_st.at[pl.ds(k * STG, STG)],
                                  aT.at[pl.ds(0, STG)], sem).wait()

        route(sem_sw, b, go)

    p1_issue(0)
    p1_issue(1)

    def _p1(ci, carry):
        b = ci % 3

        @pl.when(ci + 2 < NCH)
        def _():
            p1_issue(ci + 2)

        p1_wait(b)

        @pl.when(ci >= 3)
        def _():
            sw_wait(b)

        def _row(r, carry2):
            row = b * STG + r

            def _grp(g, carry3):
                si = src_st[row, pl.ds(g * 16, 16)]
                di = dst_st[row, pl.ds(g * 16, 16)]
                ti = et_st[row, pl.ds(g * 16, 16)]
                e = (plsc.load_gather(el_v, [si])
                     + plsc.load_gather(er_v, [di])
                     + plsc.load_gather(ee_v, [ti]))
                e = jnp.where(e > 0, e, NEG_SLOPE * e)
                sv = jnp.exp(e)
                s_st[row, pl.ds(g * 16, 16)] = sv
                plsc.addupdate_scatter(den_v, [di], sv)
                return carry3

            return lax.fori_loop(0, 8, _grp, carry2, unroll=8)

        lax.fori_loop(0, STG, _row, 0)
        sw_issue(ci, b)
        return carry

    lax.fori_loop(0, NCH, _p1, 0)
    sw_wait(2)
    sw_wait(0)
    sw_wait(1)

    pltpu.sync_copy(den_v, den_hbm.at[c * 16 + s])
    plsc.subcore_barrier()

    for qq in (1, 2, 3):
        other = c * 16 + hl * 4 + ((q + qq) % 4)

        def _piece(b, carry):
            pltpu.sync_copy(den_hbm.at[other, pl.ds(b * 2000, 2000)], tmp_v)

            def _acc(i, carry2):
                o = b * 2000 + i * 16
                den_v[pl.ds(o, 16)] = (den_v[pl.ds(o, 16)]
                                       + tmp_v[pl.ds(i * 16, 16)])
                return carry2

            return lax.fori_loop(0, 125, _acc, carry)

        lax.fori_loop(0, 5, _piece, 0)

    def p2_issue(ci):
        roff = rbase + ci * STG
        arow = h * ROWS + roff

        def go(sem, k):
            dsts = pl.ds(k * STG, STG)
            pltpu.async_copy(dst2.at[pl.ds(roff, STG)], dst_st.at[dsts], sem)
            pltpu.async_copy(aT.at[pl.ds(arow, STG)], s_st.at[dsts], sem)

        route(sem_ld, ci % 3, go)

    def p2_wait(b):
        def go(sem, k):
            dsts = pl.ds(k * STG, STG)
            pltpu.make_async_copy(dst2.at[pl.ds(0, STG)], dst_st.at[dsts],
                                  sem).wait()
            pltpu.make_async_copy(aT.at[pl.ds(0, STG)], s_st.at[dsts],
                                  sem).wait()

        route(sem_ld, b, go)

    def aw_issue(ci, b):
        arow = h * ROWS + rbase + ci * STG

        def go(sem, k):
            pltpu.async_copy(a_st.at[pl.ds(k * STG, STG)],
                             aT.at[pl.ds(arow, STG)], sem)

        route(sem_sw, b, go)

    def aw_wait(b):
        def go(sem, k):
            pltpu.make_async_copy(a_st.at[pl.ds(k * STG, STG)],
                                  aT.at[pl.ds(0, STG)], sem).wait()

        route(sem_sw, b, go)

    p2_issue(0)
    p2_issue(1)

    def _p2(ci, carry):
        b = ci % 3

        @pl.when(ci + 2 < NCH)
        def _():
            p2_issue(ci + 2)

        p2_wait(b)

        @pl.when(ci >= 3)
        def _():
            aw_wait(b)

        def _row(r, carry2):
            row = b * STG + r

            def _grp(g, carry3):
                di = dst_st[row, pl.ds(g * 16, 16)]
                dd = plsc.load_gather(den_v, [di])
                av = s_st[row, pl.ds(g * 16, 16)] / dd
                a_st[row, pl.ds(g * 16, 16)] = av
                return carry3

            return lax.fori_loop(0, 8, _grp, carry2, unroll=8)

        lax.fori_loop(0, STG, _row, 0)
        aw_issue(ci, b)
        return carry

    lax.fori_loop(0, NCH, _p2, 0)
    aw_wait(2)
    aw_wait(0)
    aw_wait(1)

    plsc.subcore_barrier()

    hh = s // 8
    p = (s // 4) % 2
    qa = s % 4
    n_ci = 63 - p
    w = p * 4 + qa

    for r in (0, 1):
        ha = c * 4 + 2 * r + hh

        @pl.when((qa == 0) & (p == 0))
        def _(ha=ha):
            pltpu.sync_copy(resT.at[ha], rst_sh.at[hh])

        plsc.subcore_barrier()

        def ag_issue(k, ha=ha):
            ci = 2 * k + p
            roff = qa * RPQ + ci * STG
            arow = ha * ROWS + roff

            def go(sem, kk):
                dsts = pl.ds(kk * STG, STG)
                pltpu.async_copy(src2.at[pl.ds(roff, STG)],
                                 src_st.at[dsts], sem)
                pltpu.async_copy(dst2.at[pl.ds(roff, STG)],
                                 dst_st.at[dsts], sem)
                pltpu.async_copy(aT.at[pl.ds(arow, STG)],
                                 a_st.at[dsts], sem)

            route(sem_ld, k % 3, go)

        def ag_wait(b):
            def go(sem, kk):
                dsts = pl.ds(kk * STG, STG)
                pltpu.make_async_copy(src2.at[pl.ds(0, STG)],
                                      src_st.at[dsts], sem).wait()
                pltpu.make_async_copy(dst2.at[pl.ds(0, STG)],
                                      dst_st.at[dsts], sem).wait()
                pltpu.make_async_copy(aT.at[pl.ds(0, STG)],
                                      a_st.at[dsts], sem).wait()

            route(sem_ld, b, go)

        def g_issue(row, x, ha=ha):
            def go(sem, kk):
                pltpu.async_copy(featT.at[ha].at[src_st.at[row]],
                                 rows4.at[kk], sem)

            route(sem_g, x, go)

        def g_wait(x, ha=ha):
            def go(sem, kk):
                pltpu.make_async_copy(featT.at[ha].at[src_st.at[0]],
                                      rows4.at[kk], sem).wait()

            route(sem_g, x, go)

        def sc_issue(row, x):
            def go(sem, kk):
                pltpu.async_copy(rows4.at[kk],
                                 rst_sh.at[hh].at[dst_st.at[row]],
                                 sem, add=True)

            route(sem_sc, x, go)

        def sc_wait(x):
            def go(sem, kk):
                pltpu.make_async_copy(rows4.at[kk],
                                      rst_sh.at[hh].at[dst_st.at[0]],
                                      sem).wait()

            route(sem_sc, x, go)

        ag_issue(0)

        @pl.when(1 < n_ci)
        def _(ag_issue=ag_issue):
            ag_issue(1)

        def _agg(k, carry, ag_issue=ag_issue, ag_wait=ag_wait,
                 g_issue=g_issue, g_wait=g_wait, sc_issue=sc_issue,
                 sc_wait=sc_wait):
            @pl.when(k < n_ci)
            def _():
                b = k % 3
                ag_wait(b)
                u0 = k * STG

                for j0 in (0, 1):
                    u = u0 + j0

                    @pl.when(u >= 4)
                    def _(u=u):
                        sc_wait(u % 4)

                    g_issue(b * STG + j0, u % 4)

                def _sub(j, carry2):
                    u = u0 + j
                    x = u % 4

                    @pl.when(j + 2 < STG)
                    def _():
                        un = u + 2

                        @pl.when(un >= 4)
                        def _():
                            sc_wait(un % 4)

                        g_issue(b * STG + j + 2, un % 4)

                    g_wait(x)

                    def _scale(i, carry3):
                        ai = plsc.load_gather(
                            a_st, [jnp.full((16,), b * STG + j, jnp.int32),
                                   jnp.full((16,), i, jnp.int32)])
                        rows4[x, i, pl.ds(0, 16)] = (
                            rows4[x, i, pl.ds(0, 16)] * ai)
                        rows4[x, i, pl.ds(16, 16)] = (
                            rows4[x, i, pl.ds(16, 16)] * ai)
                        return carry3

                    lax.fori_loop(0, 128, _scale, 0, unroll=8)
                    sc_issue(b * STG + j, x)
                    return carry2

                lax.fori_loop(0, STG, _sub, 0)

                @pl.when(k + 2 < n_ci)
                def _():
                    ag_issue(k + 2)

            return carry

        lax.fori_loop(0, 63, _agg, 0)

        n5 = n_ci * STG
        for d in range(4):
            sc_wait((n5 - 4 + d) % 4)

        plsc.subcore_barrier()

        @pl.when(w < 7)
        def _(ha=ha):
            pltpu.sync_copy(rst_sh.at[hh, pl.ds(al8(w * 1248), 1248)],
                            rstT.at[ha, pl.ds(al8(w * 1248), 1248)])

        @pl.when(w == 7)
        def _(ha=ha):
            pltpu.sync_copy(rst_sh.at[hh, pl.ds(al8(7 * 1248), N - 7 * 1248)],
                            rstT.at[ha, pl.ds(al8(7 * 1248), N - 7 * 1248)])

        plsc.subcore_barrier()


@jax.jit
def kernel(feat, edge_index, e_feat, W_fc, edge_emb, W_fc_e,
           attn_l, attn_r, attn_e, W_res):
    f32 = jnp.float32

    head_of = jnp.arange(H * OUT) // OUT
    sel = (head_of[:, None] == jnp.arange(H)[None, :]).astype(f32)
    al_m = sel * attn_l[0].reshape(H * OUT)[:, None]
    ar_m = sel * attn_r[0].reshape(H * OUT)[:, None]
    head_of_e = jnp.arange(H * EF) // EF
    sel_e = (head_of_e[:, None] == jnp.arange(H)[None, :]).astype(f32)
    ae_m = sel_e * attn_e[0].reshape(H * EF)[:, None]

    bn = 1000
    full = lambda shape: pl.BlockSpec(shape, lambda i: (0,) * len(shape))
    fs, el, er, res, ee = pl.pallas_call(
        _tc_body,
        grid=(N // bn,),
        in_specs=[
            pl.BlockSpec((bn, IN_FEATS), lambda i: (i, 0)),
            full((IN_FEATS, H * OUT)),
            full((IN_FEATS, H * OUT)),
            full((H * OUT, H)),
            full((H * OUT, H)),
            full((H, EF)),
            full((EF, H * EF)),
            full((H * EF, H)),
        ],
        out_specs=[
            pl.BlockSpec((bn, H * OUT), lambda i: (i, 0)),
            pl.BlockSpec((bn, H), lambda i: (i, 0)),
            pl.BlockSpec((bn, H), lambda i: (i, 0)),
            pl.BlockSpec((bn, H * OUT), lambda i: (i, 0)),
            full((H, H)),
        ],
        out_shape=[
            jax.ShapeDtypeStruct((N, H * OUT), f32),
            jax.ShapeDtypeStruct((N, H), f32),
            jax.ShapeDtypeStruct((N, H), f32),
            jax.ShapeDtypeStruct((N, H * OUT), f32),
            jax.ShapeDtypeStruct((H, H), f32),
        ],
    )(feat, W_fc, W_res, al_m, ar_m, edge_emb, W_fc_e, ae_m)

    src2 = edge_index[0].reshape(ROWS, 128)
    dst2 = edge_index[1].reshape(ROWS, 128)
    et2 = e_feat.reshape(ROWS, 128)
    elT = el.T.reshape(H * N)
    erT = er.T.reshape(H * N)
    eeT = jnp.pad(ee.T, ((0, 0), (0, 8))).reshape(H * 16)
    featT = fs.reshape(N, H, OUT).transpose(1, 0, 2)
    resT = res.reshape(N, H, OUT).transpose(1, 0, 2)

    mesh = plsc.VectorSubcoreMesh(core_axis_name="c", subcore_axis_name="s",
                                  num_cores=2, num_subcores=16)
    aT_rst = pl.kernel(
        _sc_body,
        out_type=[
            jax.ShapeDtypeStruct((H * ROWS, 128), f32),
            jax.ShapeDtypeStruct((H, N, OUT), f32),
            jax.ShapeDtypeStruct((32, N), f32),
        ],
        mesh=mesh,
        compiler_params=pltpu.CompilerParams(needs_layout_passes=False,
                                             use_tc_tiling_on_sc=False),
        scratch_types=[
            pltpu.VMEM((N,), f32),
            pltpu.VMEM((N,), f32),
            pltpu.VMEM((16,), f32),
            pltpu.VMEM((N,), f32),
            pltpu.VMEM((2000,), f32),
            pltpu.VMEM((3 * STG, 128), jnp.int32),
            pltpu.VMEM((3 * STG, 128), jnp.int32),
            pltpu.VMEM((3 * STG, 128), jnp.int32),
            pltpu.VMEM((3 * STG, 128), f32),
            pltpu.VMEM((3 * STG, 128), f32),
            pltpu.VMEM((4, 128, OUT), f32),
            pltpu.VMEM_SHARED((2, N, OUT), f32),
        ] + [pltpu.SemaphoreType.DMA] * 14,
    )(src2, dst2, et2, elT, erT, eeT, featT, resT)

    aT, rstT = aT_rst[0], aT_rst[1]
    a = aT.reshape(H, E).T
    rst = rstT.transpose(1, 0, 2)
    return rst, a
